# baseline (speedup 1.0000x reference)
import functools

import jax
import jax.numpy as jnp
from jax import lax
from jax.experimental import pallas as pl
from jax.experimental.pallas import tpu as pltpu
from jax.experimental.pallas import tpu_sc as plsc

B, T = 1024, 50
V, E, H = 100000, 128, 256
G = 3 * H

T1 = 20
T2 = T - T1


@functools.lru_cache(maxsize=4)
def _make_sc_gather(n_rows):
    NC, NS = 2, 16
    NW = NC * NS
    per_w = n_rows // NW
    CH = 80
    n_ch = per_w // CH

    mesh = plsc.VectorSubcoreMesh(core_axis_name="c", subcore_axis_name="s")

    @functools.partial(
        pl.kernel,
        mesh=mesh,
        out_type=jax.ShapeDtypeStruct((n_rows, E), jnp.float32),
        scratch_types=[
            pltpu.VMEM((per_w,), jnp.int32),
            pltpu.VMEM((CH, E), jnp.float32),
            pltpu.VMEM((CH, E), jnp.float32),
            pltpu.SemaphoreType.DMA,
            pltpu.SemaphoreType.DMA,
        ],
    )
    def gather_k(table_hbm, idx_hbm, out_hbm, idx_v, rows0, rows1, s0, s1):
        wid = lax.axis_index("s") * NC + lax.axis_index("c")
        base = wid * per_w
        pltpu.sync_copy(idx_hbm.at[pl.ds(base, per_w)], idx_v)

        def fire(i, buf, sem):
            pltpu.async_copy(table_hbm.at[idx_v.at[pl.ds(i * CH, CH)]], buf, sem)

        def drain(buf, sem):
            pltpu.make_async_copy(table_hbm.at[idx_v.at[pl.ds(0, CH)]], buf, sem).wait()

        fire(0, rows0, s0)
        fire(1, rows1, s1)

        def body(j, carry):
            i0 = j * 2
            drain(rows0, s0)
            pltpu.sync_copy(rows0, out_hbm.at[pl.ds(base + i0 * CH, CH)])

            @pl.when(i0 + 2 < n_ch)
            def _():
                fire(i0 + 2, rows0, s0)

            drain(rows1, s1)
            pltpu.sync_copy(rows1, out_hbm.at[pl.ds(base + (i0 + 1) * CH, CH)])

            @pl.when(i0 + 3 < n_ch)
            def _():
                fire(i0 + 3, rows1, s1)

            return carry

        lax.fori_loop(0, n_ch // 2, body, 0)

    return gather_k


NCHUNK = 4
BC = B // NCHUNK
TS = 10


def _gru_body(n_grid, has_prev, e_ref, hin_ref, *rest):
    ws = rest[:12]
    k = 12 + (1 if has_prev else 0)
    y_ref, hid_ref = rest[k], rest[k + 1]
    scr = rest[k + 2:]
    t = pl.program_id(0)
    bf = jnp.bfloat16
    h0fs = scr[0::4]
    h1fs = scr[1::4]
    a0s = scr[2::4]
    a1s = scr[3::4]

    @pl.when(t == 0)
    def _():
        for c in range(NCHUNK):
            rows = pl.ds(c * BC, BC)
            h0 = hin_ref[0, rows, :]
            h1 = hin_ref[1, rows, :]
            h0fs[c][...] = h0
            h1fs[c][...] = h1
            a0s[c][:, E:] = h0.astype(bf)
            a1s[c][:, H:] = h1.astype(bf)

    def dot(a, b):
        return lax.dot_general(a, b, (((1,), (0,)), ((), ())),
                               preferred_element_type=jnp.float32)

    def dots(a_ref, K, wrz, win, whn, brz, bin_, bhn):
        s = dot(a_ref[...], wrz[...]) + brz[...]
        gin = dot(a_ref[:, :K], win[...]) + bin_[...]
        ghn = dot(a_ref[:, K:], whn[...]) + bhn[...]
        return s, gin, ghn

    def gates(s, gin, ghn, hf_ref):
        r = 0.5 * jnp.tanh(s[:, :H]) + 0.5
        z = 0.5 * jnp.tanh(s[:, H:]) + 0.5
        n = jnp.tanh(gin + r * ghn)
        hnew = z * (hf_ref[...] - n) + n
        hf_ref[...] = hnew
        return hnew

    W0 = ws[:6]
    W1 = ws[6:]

    h0 = [None] * NCHUNK
    h1 = [None] * NCHUNK
    for u in range(TS):
        for c in range(NCHUNK):
            rows = pl.ds(c * BC, BC)
            a0s[c][:, :E] = e_ref[u, rows, :].astype(bf)

        d0 = [dots(a0s[c], E, *W0) for c in range(NCHUNK)]
        d1 = [None] * NCHUNK
        for c in range(NCHUNK):
            h0[c] = gates(*d0[c], h0fs[c])
            h0b = h0[c].astype(bf)
            a0s[c][:, E:] = h0b
            a1s[c][:, :H] = h0b
            d1[c] = dots(a1s[c], H, *W1)
        for c in range(NCHUNK):
            h1[c] = gates(*d1[c], h1fs[c])
            a1s[c][:, H:] = h1[c].astype(bf)
            y_ref[u, pl.ds(c * BC, BC), :] = h1[c]

    @pl.when(t == n_grid - 1)
    def _():
        for c in range(NCHUNK):
            rows = pl.ds(c * BC, BC)
            hid_ref[0, rows, :] = h0[c]
            hid_ref[1, rows, :] = h1[c]


def _gru_part(e_part, hin, y_prev, weights, t_off, t_len):
    n_grid = t_len // TS
    blk_off = t_off // TS
    full = lambda shape: pl.BlockSpec(shape, lambda t: tuple(0 for _ in shape))
    w_specs = [full(w.shape) for w in weights]
    has_prev = y_prev is not None
    in_specs = [
        pl.BlockSpec((TS, B, E), lambda t: (t, 0, 0)),
        full((2, B, H)),
        *w_specs,
    ]
    args = [e_part, hin, *weights]
    aliases = {}
    if has_prev:
        in_specs.append(pl.BlockSpec(memory_space=pl.ANY))
        args.append(y_prev)
        aliases = {2 + len(weights): 0}
    y, hid = pl.pallas_call(
        functools.partial(_gru_body, n_grid, has_prev),
        grid=(n_grid,),
        in_specs=in_specs,
        out_specs=[
            pl.BlockSpec((TS, B, H), lambda t: (t + blk_off, 0, 0)),
            pl.BlockSpec((2, B, H), lambda t: (0, 0, 0)),
        ],
        out_shape=[
            jax.ShapeDtypeStruct((T, B, H), jnp.float32),
            jax.ShapeDtypeStruct((2, B, H), jnp.float32),
        ],
        scratch_shapes=[
            s for _ in range(NCHUNK) for s in (
                pltpu.VMEM((BC, H), jnp.float32),
                pltpu.VMEM((BC, H), jnp.float32),
                pltpu.VMEM((BC, E + H), jnp.bfloat16),
                pltpu.VMEM((BC, 2 * H), jnp.bfloat16),
            )
        ],
        input_output_aliases=aliases,
    )(*args)
    return y, hid


def kernel(x, emb, W_ih_l0, W_hh_l0, b_ih_l0, b_hh_l0,
           W_ih_l1, W_hh_l1, b_ih_l1, b_hh_l1):
    idx = x.T.reshape(-1).astype(jnp.int32)
    e1 = _make_sc_gather(T1 * B)(emb, idx[:T1 * B]).reshape(T1, B, E)
    e2 = _make_sc_gather(T2 * B)(emb, idx[T1 * B:]).reshape(T2, B, E)

    bf = jnp.bfloat16

    def layer_weights(Wih, Whh, bih, bhh):
        wihT, whhT = Wih.T, Whh.T
        wrz = (0.5 * jnp.concatenate([wihT[:, :2 * H], whhT[:, :2 * H]],
                                     axis=0)).astype(bf)
        win = wihT[:, 2 * H:].astype(bf)
        whn = whhT[:, 2 * H:].astype(bf)
        brz = (0.5 * (bih + bhh))[:2 * H].reshape(1, 2 * H)
        bin_ = bih[2 * H:].reshape(1, H)
        bhn = bhh[2 * H:].reshape(1, H)
        return wrz, win, whn, brz, bin_, bhn

    weights = (*layer_weights(W_ih_l0, W_hh_l0, b_ih_l0, b_hh_l0),
               *layer_weights(W_ih_l1, W_hh_l1, b_ih_l1, b_hh_l1))

    hz = jnp.zeros((2, B, H), jnp.float32)
    y1, hid1 = _gru_part(e1, hz, None, weights, 0, T1)
    y, hid = _gru_part(e2, hid1, y1, weights, T1, T2)
    return jnp.swapaxes(y, 0, 1), hid
